# Initial kernel scaffold; baseline (speedup 1.0000x reference)
import jax
import jax.numpy as jnp
from jax.experimental import pallas as pl


def kernel(x):
    raise NotImplementedError("write your pallas kernel here")


# speedup vs baseline: 6.9586x; 6.9586x over previous
import functools

import jax
import jax.numpy as jnp
from jax.experimental import pallas as pl
from jax.experimental.pallas import tpu as pltpu

_ALPHA = 0.7
_KFRAC_MAX = 0.2
_KFRAC_MIN = 0.2


def _positive_k(k, n):
    if k <= 0:
        return 0
    elif k < 1:
        return int(round(float(n) * float(k)))
    elif k > n:
        return int(n)
    else:
        return int(k)


def _to_sortable(i):
    return jnp.where(i < 0, i ^ jnp.int32(0x7FFFFFFF), i)


def _from_sortable(s):
    return jnp.where(s < 0, s ^ jnp.int32(0x7FFFFFFF), s)


def _kth_largest_stats(key, x, k, negated=False):
    n, c = key.shape
    cnt0 = jnp.sum((key >= 0).astype(jnp.int32), axis=0)
    t = jnp.where(cnt0 >= k, jnp.int32(0), jnp.int32(-0x80000000))

    def body(b, t):
        bit = jnp.int32(0x40000000) >> b
        cand = t | bit
        cnt = jnp.sum((key >= cand[None, :]).astype(jnp.int32), axis=0)
        return jnp.where(cnt >= k, cand, t)

    t = jax.lax.fori_loop(0, 31, body, t, unroll=True)
    gt = key > t[None, :]
    cnt_gt = jnp.sum(gt.astype(jnp.int32), axis=0)
    sum_gt = jnp.sum(jnp.where(gt, x, jnp.float32(0.0)), axis=0)
    t_skey = ~t if negated else t
    tval = jax.lax.bitcast_convert_type(_from_sortable(t_skey), jnp.float32)
    return sum_gt + (jnp.float32(k) - cnt_gt.astype(jnp.float32)) * tval


def _pool_body(x_ref, out_ref, *, kmax, kmin):
    x = x_ref[0]
    i = jax.lax.bitcast_convert_type(x, jnp.int32)
    skey = _to_sortable(i)

    top_sum = _kth_largest_stats(skey, x, kmax)
    bot_sum = _kth_largest_stats(~skey, x, kmin, negated=True)

    res = (top_sum / jnp.float32(kmax)
           + bot_sum * jnp.float32(_ALPHA) / jnp.float32(kmin)) * jnp.float32(0.5)
    out_ref[0, 0, :] = res


def kernel(x):
    B, H, W, C = x.shape
    n = H * W
    kmax = _positive_k(_KFRAC_MAX, n)
    kmin = _positive_k(_KFRAC_MIN, n)
    xr = jnp.reshape(x, (B, n, C))

    body = functools.partial(_pool_body, kmax=kmax, kmin=kmin)
    out = pl.pallas_call(
        body,
        grid=(B,),
        in_specs=[pl.BlockSpec((1, n, C), lambda b: (b, 0, 0))],
        out_specs=pl.BlockSpec((1, 1, C), lambda b: (b, 0, 0)),
        out_shape=jax.ShapeDtypeStruct((B, 1, C), jnp.float32),
    )(xr)
    return jnp.reshape(out, (B, C))
